# Initial kernel scaffold; baseline (speedup 1.0000x reference)
import jax
import jax.numpy as jnp
from jax.experimental import pallas as pl


def kernel(phase_ids, embed_table):
    raise NotImplementedError("write your pallas kernel here")


# speedup vs baseline: 6.3454x; 6.3454x over previous
import functools

import jax
import jax.numpy as jnp
from jax import lax
from jax.experimental import pallas as pl
from jax.experimental.pallas import tpu as pltpu
from jax.experimental.pallas import tpu_sc as plsc

B = 16384
T = 200
D = 32
N = B * T
NUM_WORKERS = 32
PER_W = N // NUM_WORKERS
CHUNK = 2048
N_CHUNKS = PER_W // CHUNK

_mesh = plsc.VectorSubcoreMesh(core_axis_name="c", subcore_axis_name="s")


@functools.partial(
    pl.kernel,
    mesh=_mesh,
    out_type=jax.ShapeDtypeStruct((N, D), jnp.float32),
    scratch_types=[
        pltpu.VMEM((CHUNK,), jnp.int32),
        pltpu.VMEM((CHUNK, D), jnp.float32),
        pltpu.SemaphoreType.DMA,
    ],
    compiler_params=pltpu.CompilerParams(use_tc_tiling_on_sc=False),
)
def _gather_kernel(idx_hbm, table_hbm, out_hbm, idx_v, rows_v, sem):
    wid = lax.axis_index("s") * 2 + lax.axis_index("c")
    base = wid * PER_W

    def body(i, carry):
        off = base + i * CHUNK
        pltpu.sync_copy(idx_hbm.at[pl.ds(off, CHUNK)], idx_v)
        pltpu.async_copy(table_hbm.at[idx_v], rows_v, sem).wait()
        pltpu.sync_copy(rows_v, out_hbm.at[pl.ds(off, CHUNK)])
        return carry

    lax.fori_loop(0, N_CHUNKS, body, 0)


def kernel(phase_ids, embed_table):
    idx = phase_ids.reshape(-1).astype(jnp.int32)
    out = _gather_kernel(idx, embed_table)
    return out.reshape(phase_ids.shape + (embed_table.shape[1],))


# speedup vs baseline: 6.4950x; 1.0236x over previous
import functools

import jax
import jax.numpy as jnp
from jax import lax
from jax.experimental import pallas as pl
from jax.experimental.pallas import tpu as pltpu
from jax.experimental.pallas import tpu_sc as plsc

B = 16384
T = 200
D = 32
N = B * T
NUM_WORKERS = 32
PER_W = N // NUM_WORKERS
NBUF = 2
CHUNK = 1600
N_CHUNKS = PER_W // CHUNK
N_OUTER = N_CHUNKS // NBUF

_mesh = plsc.VectorSubcoreMesh(core_axis_name="c", subcore_axis_name="s")


@functools.partial(
    pl.kernel,
    mesh=_mesh,
    out_type=jax.ShapeDtypeStruct((N, D), jnp.float32),
    scratch_types=[
        pltpu.VMEM((NBUF, CHUNK), jnp.int32),
        pltpu.VMEM((NBUF, CHUNK, D), jnp.float32),
        pltpu.SemaphoreType.DMA,
        pltpu.SemaphoreType.DMA,
        pltpu.SemaphoreType.DMA,
        pltpu.SemaphoreType.DMA,
        pltpu.SemaphoreType.DMA,
        pltpu.SemaphoreType.DMA,
    ],
    compiler_params=pltpu.CompilerParams(use_tc_tiling_on_sc=False),
)
def _gather_kernel(idx_hbm, table_hbm, out_hbm, idx_v, rows_v,
                   si0, si1, sg0, sg1, ss0, ss1):
    sem_i = [si0, si1]
    sem_g = [sg0, sg1]
    sem_s = [ss0, ss1]
    wid = lax.axis_index("s") * 2 + lax.axis_index("c")
    base = wid * PER_W

    for b in range(NBUF):
        pltpu.async_copy(idx_hbm.at[pl.ds(base + b * CHUNK, CHUNK)],
                         idx_v.at[b], sem_i[b])

    def outer(g, carry):
        for b in range(NBUF):
            off = base + (g * NBUF + b) * CHUNK
            pltpu.make_async_copy(idx_hbm.at[pl.ds(off, CHUNK)],
                                  idx_v.at[b], sem_i[b]).wait()

            @pl.when(g >= 1)
            def _():
                pltpu.make_async_copy(rows_v.at[b],
                                      out_hbm.at[pl.ds(off, CHUNK)],
                                      sem_s[b]).wait()

            pltpu.async_copy(table_hbm.at[idx_v.at[b]], rows_v.at[b],
                             sem_g[b]).wait()
            pltpu.async_copy(rows_v.at[b], out_hbm.at[pl.ds(off, CHUNK)],
                             sem_s[b])

            @pl.when(g + 1 < N_OUTER)
            def _():
                pltpu.async_copy(idx_hbm.at[pl.ds(off + NBUF * CHUNK, CHUNK)],
                                 idx_v.at[b], sem_i[b])
        return carry

    lax.fori_loop(0, N_OUTER, outer, 0)

    for b in range(NBUF):
        last_off = base + ((N_OUTER - 1) * NBUF + b) * CHUNK
        pltpu.make_async_copy(rows_v.at[b], out_hbm.at[pl.ds(last_off, CHUNK)],
                              sem_s[b]).wait()


def kernel(phase_ids, embed_table):
    idx = phase_ids.reshape(-1).astype(jnp.int32)
    out = _gather_kernel(idx, embed_table)
    return out.reshape(phase_ids.shape + (embed_table.shape[1],))


# speedup vs baseline: 6.5080x; 1.0020x over previous
import functools

import jax
import jax.numpy as jnp
from jax import lax
from jax.experimental import pallas as pl
from jax.experimental.pallas import tpu as pltpu
from jax.experimental.pallas import tpu_sc as plsc

B = 16384
T = 200
D = 32
N = B * T
NUM_WORKERS = 32
PER_W = N // NUM_WORKERS
NBUF = 4
CHUNK = 800
N_CHUNKS = PER_W // CHUNK
N_OUTER = N_CHUNKS // NBUF

_mesh = plsc.VectorSubcoreMesh(core_axis_name="c", subcore_axis_name="s")


@functools.partial(
    pl.kernel,
    mesh=_mesh,
    out_type=jax.ShapeDtypeStruct((N, D), jnp.float32),
    scratch_types=[
        pltpu.VMEM((NBUF, CHUNK), jnp.int32),
        pltpu.VMEM((NBUF, CHUNK, D), jnp.float32),
        [pltpu.SemaphoreType.DMA] * NBUF,
        [pltpu.SemaphoreType.DMA] * NBUF,
        [pltpu.SemaphoreType.DMA] * NBUF,
    ],
    compiler_params=pltpu.CompilerParams(use_tc_tiling_on_sc=False),
)
def _gather_kernel(idx_hbm, table_hbm, out_hbm, idx_v, rows_v,
                   sem_i, sem_g, sem_s):
    wid = lax.axis_index("s") * 2 + lax.axis_index("c")
    base = wid * PER_W

    def wait_idx(i, b):
        pltpu.make_async_copy(idx_hbm.at[pl.ds(base, CHUNK)],
                              idx_v.at[b], sem_i[b]).wait()

    def wait_store(b):
        pltpu.make_async_copy(rows_v.at[b], out_hbm.at[pl.ds(base, CHUNK)],
                              sem_s[b]).wait()

    def wait_gather(b):
        pltpu.make_async_copy(table_hbm.at[idx_v.at[b]], rows_v.at[b],
                              sem_g[b]).wait()

    for b in range(NBUF):
        pltpu.async_copy(idx_hbm.at[pl.ds(base + b * CHUNK, CHUNK)],
                         idx_v.at[b], sem_i[b])

    def outer(g, carry):
        for b in range(NBUF):
            i = g * NBUF + b
            off = base + i * CHUNK
            bp = (b - 1) % NBUF

            wait_idx(i, b)

            @pl.when(g >= 1)
            def _():
                wait_store(b)

            pltpu.async_copy(table_hbm.at[idx_v.at[b]], rows_v.at[b],
                             sem_g[b])

            @pl.when(i >= 1)
            def _():
                wait_gather(bp)
                pltpu.async_copy(rows_v.at[bp],
                                 out_hbm.at[pl.ds(off - CHUNK, CHUNK)],
                                 sem_s[bp])

            @pl.when((i >= 1) & (i - 1 + NBUF < N_CHUNKS))
            def _():
                pltpu.async_copy(
                    idx_hbm.at[pl.ds(off - CHUNK + NBUF * CHUNK, CHUNK)],
                    idx_v.at[bp], sem_i[bp])
        return carry

    lax.fori_loop(0, N_OUTER, outer, 0)

    last = N_CHUNKS - 1
    bl = last % NBUF
    wait_gather(bl)
    pltpu.async_copy(rows_v.at[bl],
                     out_hbm.at[pl.ds(base + last * CHUNK, CHUNK)],
                     sem_s[bl])
    for b in range(NBUF):
        wait_store(b)


def kernel(phase_ids, embed_table):
    idx = phase_ids.reshape(-1).astype(jnp.int32)
    out = _gather_kernel(idx, embed_table)
    return out.reshape(phase_ids.shape + (embed_table.shape[1],))
